# baseline (speedup 1.0000x reference)
import jax
import jax.numpy as jnp
from jax.experimental import pallas as pl
from jax.experimental.pallas import tpu as pltpu

IN_F = 768
OUT_F = 768
RANK = 8
N_EXP = 4
N_TOK = 4 * 2048

ROW_TILE = 2048
NT = N_TOK // ROW_TILE


def _body(x_ref, r_ref, wt_ref, d_ref, u_ref, o_ref, wc_ref):
    j = pl.program_id(0)

    @pl.when(j == 0)
    def _combine():
        s = jnp.sum(x_ref[...], axis=0, keepdims=True)
        om = jnp.dot(s * (1.0 / N_TOK), r_ref[...],
                     preferred_element_type=jnp.float32)
        o0, o1, o2, o3 = om[0, 0], om[0, 1], om[0, 2], om[0, 3]
        mx = jnp.maximum(jnp.maximum(o0, o1), jnp.maximum(o2, o3))
        e0 = jnp.exp(o0 - mx)
        e1 = jnp.exp(o1 - mx)
        e2 = jnp.exp(o2 - mx)
        e3 = jnp.exp(o3 - mx)
        z = e0 + e1 + e2 + e3
        idx = jax.lax.broadcasted_iota(jnp.int32, (1, N_EXP * RANK), 1) // RANK
        gcol = jnp.where(idx == 0, e0,
                         jnp.where(idx == 1, e1,
                                   jnp.where(idx == 2, e2, e3))) / z
        wc = wt_ref[...] + jnp.dot(
            d_ref[...] * gcol, u_ref[...], preferred_element_type=jnp.float32)
        wc_ref[...] = wc.astype(jnp.bfloat16)

    xb16 = x_ref[pl.ds(j * ROW_TILE, ROW_TILE), :].astype(jnp.bfloat16)
    o_ref[...] = jnp.dot(xb16, wc_ref[...], preferred_element_type=jnp.float32)


@jax.jit
def _run(x2, route_all, wt, dcat, ucat):
    return pl.pallas_call(
        _body,
        grid=(NT,),
        in_specs=[
            pl.BlockSpec((N_TOK, IN_F), lambda i: (0, 0)),
            pl.BlockSpec(route_all.shape, lambda i: (0, 0)),
            pl.BlockSpec(wt.shape, lambda i: (0, 0)),
            pl.BlockSpec(dcat.shape, lambda i: (0, 0)),
            pl.BlockSpec(ucat.shape, lambda i: (0, 0)),
        ],
        out_specs=pl.BlockSpec((ROW_TILE, OUT_F), lambda i: (i, 0)),
        out_shape=jax.ShapeDtypeStruct((N_TOK, OUT_F), jnp.float32),
        scratch_shapes=[
            pltpu.VMEM((IN_F, OUT_F), jnp.bfloat16),
        ],
    )(x2, route_all, wt, dcat, ucat)


def kernel(input, task_id, W, lora_down, lora_up, lora_route):
    B, S, F = input.shape
    x2 = input.reshape(B * S, F)
    route_all = lora_route[0] + lora_route[1] + lora_route[2]
    wt = W.T
    dcat = jnp.transpose(lora_down[:N_EXP], (1, 0, 2)).reshape(F, N_EXP * RANK)
    ucat = lora_up[:N_EXP].reshape(N_EXP * RANK, OUT_F)
    out = _run(x2, route_all, wt, dcat, ucat)
    return out.reshape(B, S, OUT_F)
